# baseline (speedup 1.0000x reference)
import functools

import jax
import jax.numpy as jnp
from jax import lax
from jax.experimental import pallas as pl
from jax.experimental.pallas import tpu as pltpu
from jax.experimental.pallas import tpu_sc as plsc

ROWS = 128
COLS = 32768
LANES = 16
NUM_WORKERS = 32
ROWS_PER_WORKER = ROWS // NUM_WORKERS
UNROLL = 8
STRIDE = UNROLL * LANES

_GATHER_DNUMS = lax.GatherDimensionNumbers(
    offset_dims=(), collapsed_slice_dims=(0,), start_index_map=(0,))


def _lane_gather(x, perm):
    return lax.gather(
        x, perm[:, None], _GATHER_DNUMS, slice_sizes=(1,),
        mode=lax.GatherScatterMode.PROMISE_IN_BOUNDS)


def _merge(va, ia, vb, ib):
    take = (vb > va) | ((vb == va) & (ib < ia))
    return jnp.where(take, vb, va), jnp.where(take, ib, ia)


def _argmax_body(logits_hbm, out_hbm, buf0, buf1, sem0, sem1, res_v):
    cid = lax.axis_index("c")
    sid = lax.axis_index("s")
    wid = sid * 2 + cid
    base_row = wid * ROWS_PER_WORKER

    bufs = (buf0, buf1)
    sems = (sem0, sem1)
    copies = [None, None]
    copies[0] = pltpu.async_copy(
        logits_hbm.at[base_row], buf0.at[pl.ds(0, COLS)], sem0)

    iota = lax.broadcasted_iota(jnp.int32, (LANES,), 0)
    n_iters = COLS // STRIDE

    for r in range(ROWS_PER_WORKER):
        if r + 1 < ROWS_PER_WORKER:
            copies[(r + 1) % 2] = pltpu.async_copy(
                logits_hbm.at[base_row + r + 1],
                bufs[(r + 1) % 2].at[pl.ds(0, COLS)],
                sems[(r + 1) % 2])
        copies[r % 2].wait()
        buf = bufs[r % 2]

        xs0 = tuple(buf[pl.ds(u * LANES, LANES)] for u in range(UNROLL))
        init = (tuple(jnp.full((LANES,), -jnp.inf, jnp.float32)
                      for _ in range(UNROLL)),
                tuple(iota for _ in range(UNROLL)),
                xs0)

        @plsc.parallel_loop(0, n_iters, 1, carry=init)
        def body(i, carry, buf=buf):
            bests, iters, xs = carry
            base = (i + 1) * STRIDE
            new_xs = tuple(buf[pl.ds(base + u * LANES, LANES)]
                           for u in range(UNROLL))
            new_bests, new_iters = [], []
            for u in range(UNROLL):
                m = xs[u] > bests[u]
                new_bests.append(jnp.maximum(bests[u], xs[u]))
                new_iters.append(jnp.where(m, i, iters[u]))
            return tuple(new_bests), tuple(new_iters), new_xs

        bests, iters, _ = body

        best, bidx = None, None
        for u in range(UNROLL):
            idx_u = iters[u] * STRIDE + (u * LANES + iota)
            if best is None:
                best, bidx = bests[u], idx_u
            else:
                best, bidx = _merge(best, bidx, bests[u], idx_u)

        for shift in (8, 4, 2, 1):
            perm = iota ^ shift
            oval = _lane_gather(best, perm)
            oidx = _lane_gather(bidx, perm)
            best, bidx = _merge(best, bidx, oval, oidx)
        res_v[r] = bidx

    pltpu.sync_copy(res_v, out_hbm.at[pl.ds(base_row, ROWS_PER_WORKER)])


@functools.partial(
    pl.kernel,
    out_type=jax.ShapeDtypeStruct((ROWS, LANES), jnp.int32),
    mesh=plsc.VectorSubcoreMesh(core_axis_name="c", subcore_axis_name="s"),
    scratch_types=[
        pltpu.VMEM((COLS + STRIDE,), jnp.float32),
        pltpu.VMEM((COLS + STRIDE,), jnp.float32),
        pltpu.SemaphoreType.DMA,
        pltpu.SemaphoreType.DMA,
        pltpu.VMEM((ROWS_PER_WORKER, LANES), jnp.int32),
    ],
)
def _sc_argmax(logits_hbm, out_hbm, buf0, buf1, sem0, sem1, res_v):
    _argmax_body(logits_hbm, out_hbm, buf0, buf1, sem0, sem1, res_v)


def kernel(logits):
    out = _sc_argmax(logits)
    return out[:, :1]


# speedup vs baseline: 1.0142x; 1.0142x over previous
import functools

import jax
import jax.numpy as jnp
from jax import lax
from jax.experimental import pallas as pl
from jax.experimental.pallas import tpu as pltpu
from jax.experimental.pallas import tpu_sc as plsc

ROWS = 128
COLS = 32768
LANES = 16
NUM_WORKERS = 32
ROWS_PER_WORKER = ROWS // NUM_WORKERS
UNROLL = 8
STRIDE = UNROLL * LANES

_GATHER_DNUMS = lax.GatherDimensionNumbers(
    offset_dims=(), collapsed_slice_dims=(0,), start_index_map=(0,))


def _lane_gather(x, perm):
    return lax.gather(
        x, perm[:, None], _GATHER_DNUMS, slice_sizes=(1,),
        mode=lax.GatherScatterMode.PROMISE_IN_BOUNDS)


def _merge(va, ia, vb, ib):
    take = (vb > va) | ((vb == va) & (ib < ia))
    return jnp.where(take, vb, va), jnp.where(take, ib, ia)


def _argmax_body(logits_hbm, out_hbm, buf0, buf1, sem0, sem1, res_v):
    cid = lax.axis_index("c")
    sid = lax.axis_index("s")
    wid = sid * 2 + cid
    base_row = wid * ROWS_PER_WORKER

    bufs = (buf0, buf1)
    sems = (sem0, sem1)
    copies = [None, None]
    copies[0] = pltpu.async_copy(
        logits_hbm.at[base_row], buf0.at[pl.ds(0, COLS)], sem0)

    iota = lax.broadcasted_iota(jnp.int32, (LANES,), 0)
    n_iters = COLS // STRIDE

    for r in range(ROWS_PER_WORKER):
        if r + 1 < ROWS_PER_WORKER:
            copies[(r + 1) % 2] = pltpu.async_copy(
                logits_hbm.at[base_row + r + 1],
                bufs[(r + 1) % 2].at[pl.ds(0, COLS)],
                sems[(r + 1) % 2])
        copies[r % 2].wait()
        buf = bufs[r % 2]

        xs0 = tuple(buf[pl.ds(u * LANES, LANES)] for u in range(UNROLL))
        init = (tuple(jnp.full((LANES,), -jnp.inf, jnp.float32)
                      for _ in range(UNROLL)),
                tuple(iota for _ in range(UNROLL)),
                xs0)

        @plsc.parallel_loop(0, n_iters, 1, carry=init)
        def body(i, carry, buf=buf):
            bests, iters, xs = carry
            base = (i + 1) * STRIDE
            new_xs = tuple(buf[pl.ds(base + u * LANES, LANES)]
                           for u in range(UNROLL))
            new_bests, new_iters = [], []
            for u in range(UNROLL):
                m = xs[u] > bests[u]
                new_bests.append(jnp.maximum(bests[u], xs[u]))
                new_iters.append(jnp.where(m, i, iters[u]))
            return tuple(new_bests), tuple(new_iters), new_xs

        bests, iters, _ = body

        best, bidx = None, None
        for u in range(UNROLL):
            idx_u = iters[u] * STRIDE + (u * LANES + iota)
            if best is None:
                best, bidx = bests[u], idx_u
            else:
                best, bidx = _merge(best, bidx, bests[u], idx_u)

        for shift in (8, 4, 2, 1):
            perm = iota ^ shift
            oval = _lane_gather(best, perm)
            oidx = _lane_gather(bidx, perm)
            best, bidx = _merge(best, bidx, oval, oidx)
        res_v[r] = bidx

    pltpu.sync_copy(res_v, out_hbm.at[pl.ds(base_row, ROWS_PER_WORKER)])


@functools.partial(
    pl.kernel,
    out_type=jax.ShapeDtypeStruct((ROWS, LANES), jnp.int32),
    mesh=plsc.VectorSubcoreMesh(core_axis_name="c", subcore_axis_name="s"),
    scratch_types=[
        pltpu.VMEM((COLS + STRIDE,), jnp.float32),
        pltpu.VMEM((COLS + STRIDE,), jnp.float32),
        pltpu.SemaphoreType.DMA,
        pltpu.SemaphoreType.DMA,
        pltpu.VMEM((ROWS_PER_WORKER, LANES), jnp.int32),
    ],
)
def _sc_argmax(logits_hbm, out_hbm, buf0, buf1, sem0, sem1, res_v):
    _argmax_body(logits_hbm, out_hbm, buf0, buf1, sem0, sem1, res_v)


TC_ROW_BLOCK = 8
TC_CHUNK = 128


def _tc_argmax_body(x_ref, out_ref):
    n_chunks = COLS // TC_CHUNK

    def body(c, carry):
        best, it = carry
        x = x_ref[:, pl.ds(c * TC_CHUNK, TC_CHUNK)]
        m = x > best
        return jnp.maximum(best, x), jnp.where(m, c, it)

    init = (jnp.full((TC_ROW_BLOCK, TC_CHUNK), -jnp.inf, jnp.float32),
            jnp.zeros((TC_ROW_BLOCK, TC_CHUNK), jnp.int32))
    best, it = lax.fori_loop(0, n_chunks, body, init)

    lane = lax.broadcasted_iota(jnp.int32, (TC_ROW_BLOCK, TC_CHUNK), 1)
    idx = it * TC_CHUNK + lane
    row_max = jnp.max(best, axis=-1, keepdims=True)
    cand = jnp.where(best == row_max, idx, jnp.int32(2147483647))
    out_ref[...] = jnp.min(cand, axis=-1, keepdims=True)


def _tc_argmax(x):
    rows = x.shape[0]
    return pl.pallas_call(
        _tc_argmax_body,
        grid=(rows // TC_ROW_BLOCK,),
        in_specs=[pl.BlockSpec((TC_ROW_BLOCK, COLS), lambda i: (i, 0))],
        out_specs=pl.BlockSpec((TC_ROW_BLOCK, 1), lambda i: (i, 0)),
        out_shape=jax.ShapeDtypeStruct((rows, 1), jnp.int32),
    )(x)


def kernel(logits):
    return _tc_argmax(logits)
